# baseline (speedup 1.0000x reference)
import functools

import jax
import jax.numpy as jnp
from jax import lax
from jax.experimental import pallas as pl
from jax.experimental.pallas import tpu as pltpu
from jax.experimental.pallas import tpu_sc as plsc

N = 10000
E = 320000
H = 8
D = 128
DV = D // H
RPD = 32
RDV = RPD // H
VOCAB = 21
EPS = 1e-12

NC = 2
NS = 16
L = 16
HG = H // NC

ROWW = 80
ACCW = 144
EPT = E // NS
C = 80
NCHUNK = EPT // C
SUP = 5
NSUPER = NCHUNK // SUP
NPT = N // NS


def _leaky(a):
    return jnp.where(a > 0, a, 0.2 * a)


PBLK = 2000


def _prep_body(x_ref, wv_ref, ws_ref, wt_ref, vall_ref, st_ref):
    v2 = jnp.dot(x_ref[...], wv_ref[...], preferred_element_type=jnp.float32)
    v3 = v2.reshape(PBLK, H, DV)
    s_src = (v3 * ws_ref[...]).sum(-1)
    s_tgt = (v3 * wt_ref[...]).sum(-1)
    z12 = jnp.zeros((PBLK, 12), jnp.float32)
    vall_ref[...] = jnp.stack([
        jnp.concatenate([v2[:, :64], s_src[:, 0:4], z12], axis=1),
        jnp.concatenate([v2[:, 64:], s_src[:, 4:8], z12], axis=1)], axis=0)
    st_ref[...] = jnp.concatenate([s_tgt, s_src], axis=1)


def _tbl_body(re_ref, wr_ref, wrel_ref, et_ref):
    r_tbl = jnp.dot(re_ref[...], wr_ref[...], preferred_element_type=jnp.float32)
    sr = _leaky((r_tbl.reshape(VOCAB, H, RDV) * wrel_ref[...]).sum(-1))
    etbl = jnp.exp(sr - jnp.max(sr))
    et_ref[...] = jnp.concatenate(
        [etbl, jnp.zeros((32 - VOCAB, H), jnp.float32)], axis=0)


def _edge_body(vall_hbm, sttgt_hbm, et_hbm, sp_hbm, tg_hbm,
               out_hbm,
               sb_sp0, sb_sp1, sb_tg0, sb_tg1, srcp0, srcp1,
               rows0, rows1, trow0, trow1, con0, et_v, acc,
               isem0, isem1, gsem0, gsem1):
    c = lax.axis_index("c")
    s = lax.axis_index("s")
    SB_SP = (sb_sp0, sb_sp1)
    SB_TG = (sb_tg0, sb_tg1)
    SRCP = (srcp0, srcp1)
    ROWS = (rows0, rows1)
    TROW = (trow0, trow1)
    ISEM = (isem0, isem1)
    GSEM = (gsem0, gsem1)

    pltpu.sync_copy(et_hbm, et_v)
    zero = jnp.zeros((L,), jnp.float32)
    lanes = lax.iota(jnp.int32, L)
    cN = c * N

    def _zc(i, carry):
        for k in range(ACCW // L):
            con0[i, pl.ds(k * L, L)] = zero
        return carry
    lax.fori_loop(0, C, _zc, 0)

    nb = s * NPT
    for k in range(NPT // C):
        pltpu.sync_copy(con0, acc.at[pl.ds(nb + k * C, C)])
    rem = NPT % C
    if rem:
        pltpu.sync_copy(con0.at[pl.ds(0, rem)],
                        acc.at[pl.ds(nb + (NPT // C) * C, rem)])
    plsc.subcore_barrier()

    def fire_idx(S, ib):
        pltpu.async_copy(sp_hbm.at[s, S], SB_SP[ib], ISEM[ib])
        pltpu.async_copy(tg_hbm.at[s, S], SB_TG[ib], ISEM[ib])

    def wait_idx(ib):
        pltpu.make_async_copy(sp_hbm.at[s, 0], SB_SP[ib], ISEM[ib]).wait()
        pltpu.make_async_copy(tg_hbm.at[s, 0], SB_TG[ib], ISEM[ib]).wait()

    def fire_gath(ib, jj, b):
        for k in range(C // L):
            sl = pl.ds(k * L, L)
            SRCP[b][sl] = (SB_SP[ib][jj, sl] >> 5) + cN
        pltpu.async_copy(vall_hbm.at[SRCP[b]], ROWS[b], GSEM[b])
        pltpu.async_copy(sttgt_hbm.at[SB_TG[ib].at[jj]], TROW[b], GSEM[b])

    def wait_gath(b):
        pltpu.make_async_copy(vall_hbm.at[SRCP[b]], ROWS[b], GSEM[b]).wait()
        pltpu.make_async_copy(sttgt_hbm.at[SB_TG[0].at[0]], TROW[b], GSEM[b]).wait()

    def compute(ib, jj, b):
        rows = ROWS[b]
        trow = TROW[b]
        con = con0

        def gg_body(gg, carry):
            elanes = lanes + gg * L
            spack16 = SB_SP[ib][jj, pl.ds(gg * L, L)]
            rel16 = spack16 & 31
            for h in range(HG):
                hcol = jnp.zeros((L,), jnp.int32) + (c * HG + h)
                a = plsc.load_gather(
                    rows, [elanes, jnp.full((L,), 64 + h, jnp.int32)])
                b_ = plsc.load_gather(trow, [elanes, hcol])
                sv = a + b_
                sv = jnp.where(sv > 0, sv, 0.2 * sv)
                w1 = jnp.exp(sv)
                w2 = plsc.load_gather(et_v, [rel16, hcol])
                plsc.store_scatter(
                    con, [elanes, jnp.full((L,), 128 + h, jnp.int32)], w1)
                plsc.store_scatter(
                    con, [elanes, jnp.full((L,), 132 + h, jnp.int32)], w2)
                for dd in range(DV):
                    d = h * DV + dd
                    vals = plsc.load_gather(
                        rows, [elanes, jnp.full((L,), d, jnp.int32)])
                    plsc.store_scatter(
                        con, [elanes, jnp.full((L,), d, jnp.int32)], w1 * vals)
                    plsc.store_scatter(
                        con, [elanes, jnp.full((L,), 64 + d, jnp.int32)], w2 * vals)
            return carry

        lax.fori_loop(0, C // L, gg_body, 0)

    fire_idx(0, 0)
    wait_idx(0)
    fire_gath(0, 0, 0)

    def outer(t, carry):
        S0 = t * 2
        for j in range(2 * SUP):
            ib, jj = (0, j) if j < SUP else (1, j - SUP)
            b = j % 2
            wait_gath(b)
            if j == 1:
                fire_idx(S0 + 1, 1)
            if j == SUP - 1:
                wait_idx(1)
            if j == SUP + 1:
                fire_idx(jnp.minimum(S0 + 2, NSUPER - 1), 0)
            if j == 2 * SUP - 1:
                wait_idx(0)
            nib, njj = (ib, jj + 1) if jj + 1 < SUP else ((1, 0) if ib == 0 else (0, 0))
            fire_gath(nib, njj, 1 - b)
            compute(ib, jj, b)
            pltpu.sync_copy(con0, acc.at[SB_TG[ib].at[jj]], add=True)
        return carry
    lax.fori_loop(0, NCHUNK // (2 * SUP), outer, 0)

    wait_gath(0)
    plsc.subcore_barrier()

    for k in range(NPT // C):
        pltpu.sync_copy(acc.at[pl.ds(nb + k * C, C)],
                        out_hbm.at[c, pl.ds(nb + k * C, C)])
    if rem:
        pltpu.sync_copy(acc.at[pl.ds(nb + (NPT // C) * C, rem)],
                        out_hbm.at[c, pl.ds(nb + (NPT // C) * C, rem)])


_edge_kernel = functools.partial(
    pl.kernel,
    out_type=jax.ShapeDtypeStruct((NC, N, ACCW), jnp.float32),
    mesh=plsc.VectorSubcoreMesh(core_axis_name="c", subcore_axis_name="s"),
    scratch_types=[
        pltpu.VMEM((SUP, C), jnp.int32),
        pltpu.VMEM((SUP, C), jnp.int32),
        pltpu.VMEM((SUP, C), jnp.int32),
        pltpu.VMEM((SUP, C), jnp.int32),
        pltpu.VMEM((C,), jnp.int32),
        pltpu.VMEM((C,), jnp.int32),
        pltpu.VMEM((C, ROWW), jnp.float32),
        pltpu.VMEM((C, ROWW), jnp.float32),
        pltpu.VMEM((C, 16), jnp.float32),
        pltpu.VMEM((C, 16), jnp.float32),
        pltpu.VMEM((C, ACCW), jnp.float32),
        pltpu.VMEM((32, H), jnp.float32),
        pltpu.VMEM_SHARED((N, ACCW), jnp.float32),
        pltpu.SemaphoreType.DMA,
        pltpu.SemaphoreType.DMA,
        pltpu.SemaphoreType.DMA,
        pltpu.SemaphoreType.DMA,
    ],
    compiler_params=pltpu.CompilerParams(use_tc_tiling_on_sc=False,
                                         needs_layout_passes=False),
)(_edge_body)


def _final_body(acc_ref, fpw_ref, fpb_ref, out_ref):
    a = acc_ref[...]
    parts = []
    for hh in range(H):
        cc, j = hh // HG, hh % HG
        den_v = a[cc, :, 128 + j:129 + j] + EPS
        den_r = a[cc, :, 132 + j:133 + j] + EPS
        parts.append(a[cc, :, j * 16:(j + 1) * 16] / den_v)
        parts.append(a[cc, :, 64 + j * 16:64 + (j + 1) * 16] / den_r)
    cat = jnp.concatenate(parts, axis=1)
    out_ref[...] = (jnp.dot(cat, fpw_ref[...], preferred_element_type=jnp.float32)
                    + fpb_ref[...])


def kernel(x, edge_index, rel_pos_idx, W_value, rel_emb, W_relation,
           w_src, w_tgt, w_rel, fp_w, fp_b):
    spack = (edge_index[0] * 32 + rel_pos_idx).reshape(NS, NSUPER, SUP, C)
    tgt_r = edge_index[1].reshape(NS, NSUPER, SUP, C)

    v_all3, st_tgt = pl.pallas_call(
        _prep_body,
        grid=(N // PBLK,),
        in_specs=[
            pl.BlockSpec((PBLK, D), lambda i: (i, 0)),
            pl.BlockSpec((D, D), lambda i: (0, 0)),
            pl.BlockSpec((1, H, DV), lambda i: (0, 0, 0)),
            pl.BlockSpec((1, H, DV), lambda i: (0, 0, 0)),
        ],
        out_specs=[
            pl.BlockSpec((2, PBLK, ROWW), lambda i: (0, i, 0)),
            pl.BlockSpec((PBLK, 16), lambda i: (i, 0)),
        ],
        out_shape=[
            jax.ShapeDtypeStruct((2, N, ROWW), jnp.float32),
            jax.ShapeDtypeStruct((N, 16), jnp.float32),
        ],
    )(x, W_value, w_src, w_tgt)
    v_all = v_all3.reshape(2 * N, ROWW)

    et = pl.pallas_call(
        _tbl_body,
        out_shape=jax.ShapeDtypeStruct((32, H), jnp.float32),
    )(rel_emb, W_relation, w_rel)

    acc2 = _edge_kernel(v_all, st_tgt, et, spack, tgt_r)

    eye = jnp.eye(H, dtype=jnp.float32)
    fpw_bd = (eye[:, None, :, None] * fp_w[:, :, None, :]).reshape(H * 2 * DV, H * DV)
    fpb_flat = fp_b.reshape(1, H * DV)

    BLK = 2000
    return pl.pallas_call(
        _final_body,
        grid=(N // BLK,),
        in_specs=[
            pl.BlockSpec((NC, BLK, ACCW), lambda i: (0, i, 0)),
            pl.BlockSpec((H * 2 * DV, H * DV), lambda i: (0, 0)),
            pl.BlockSpec((1, H * DV), lambda i: (0, 0)),
        ],
        out_specs=pl.BlockSpec((BLK, H * DV), lambda i: (i, 0)),
        out_shape=jax.ShapeDtypeStruct((N, H * DV), jnp.float32),
    )(acc2, fpw_bd, fpb_flat)


# speedup vs baseline: 1.0105x; 1.0105x over previous
import functools

import jax
import jax.numpy as jnp
from jax import lax
from jax.experimental import pallas as pl
from jax.experimental.pallas import tpu as pltpu
from jax.experimental.pallas import tpu_sc as plsc

N = 10000
E = 320000
H = 8
D = 128
DV = D // H
RPD = 32
RDV = RPD // H
VOCAB = 21
EPS = 1e-12

NC = 2
NS = 16
L = 16
HG = H // NC

ROWW = 80
ACCW = 144
EPT = E // NS
C = 80
NCHUNK = EPT // C
SUP = 5
NSUPER = NCHUNK // SUP
NPT = N // NS


def _leaky(a):
    return jnp.where(a > 0, a, 0.2 * a)


PBLK = 2000


def _prep_body(x_ref, wv_ref, ws_ref, wt_ref, vall_ref, st_ref):
    v2 = jnp.dot(x_ref[...], wv_ref[...], preferred_element_type=jnp.float32)
    v3 = v2.reshape(PBLK, H, DV)
    s_src = (v3 * ws_ref[...]).sum(-1)
    s_tgt = (v3 * wt_ref[...]).sum(-1)
    z12 = jnp.zeros((PBLK, 12), jnp.float32)
    vall_ref[...] = jnp.stack([
        jnp.concatenate([v2[:, :64], s_src[:, 0:4], z12], axis=1),
        jnp.concatenate([v2[:, 64:], s_src[:, 4:8], z12], axis=1)], axis=0)
    st_ref[...] = jnp.concatenate([s_tgt, s_src], axis=1)


def _tbl_body(re_ref, wr_ref, wrel_ref, et_ref):
    r_tbl = jnp.dot(re_ref[...], wr_ref[...], preferred_element_type=jnp.float32)
    sr = _leaky((r_tbl.reshape(VOCAB, H, RDV) * wrel_ref[...]).sum(-1))
    etbl = jnp.exp(sr - jnp.max(sr))
    et_ref[...] = jnp.concatenate(
        [etbl, jnp.zeros((32 - VOCAB, H), jnp.float32)], axis=0)


def _edge_body(vall_hbm, sttgt_hbm, et_hbm, sp_hbm, tg_hbm,
               out_hbm,
               sb_sp, sb_tg, srcp, rows, trow, con0, et_v, acc,
               isem, gsem):
    c = lax.axis_index("c")
    s = lax.axis_index("s")

    pltpu.sync_copy(et_hbm, et_v)
    zero = jnp.zeros((L,), jnp.float32)
    lanes = lax.iota(jnp.int32, L)
    cN = c * N

    def _zc(i, carry):
        for k in range(ACCW // L):
            con0[i, pl.ds(k * L, L)] = zero
        return carry
    lax.fori_loop(0, C, _zc, 0)

    nb = s * NPT
    for k in range(NPT // C):
        pltpu.sync_copy(con0, acc.at[pl.ds(nb + k * C, C)])
    rem = NPT % C
    if rem:
        pltpu.sync_copy(con0.at[pl.ds(0, rem)],
                        acc.at[pl.ds(nb + (NPT // C) * C, rem)])
    plsc.subcore_barrier()

    def fire_idx(S, half):
        hb = half * SUP
        pltpu.async_copy(sp_hbm.at[s, S], sb_sp.at[pl.ds(hb, SUP)], isem)
        pltpu.async_copy(tg_hbm.at[s, S], sb_tg.at[pl.ds(hb, SUP)], isem)

    def wait_idx():
        pltpu.make_async_copy(sp_hbm.at[s, 0], sb_sp.at[pl.ds(0, SUP)], isem).wait()
        pltpu.make_async_copy(tg_hbm.at[s, 0], sb_tg.at[pl.ds(0, SUP)], isem).wait()

    def fire_gath(gn):
        njj = lax.rem(gn, 2 * SUP)
        nb2 = lax.rem(gn, 2)
        nbase = nb2 * C
        for k in range(C // L):
            sl = pl.ds(k * L, L)
            srcp[nb2, sl] = (sb_sp[njj, sl] >> 5) + cN
        pltpu.async_copy(vall_hbm.at[srcp.at[nb2]],
                         rows.at[pl.ds(nbase, C)], gsem)
        pltpu.async_copy(sttgt_hbm.at[sb_tg.at[njj]],
                         trow.at[pl.ds(nbase, C)], gsem)

    def wait_gath():
        pltpu.make_async_copy(vall_hbm.at[srcp.at[0]],
                              rows.at[pl.ds(0, C)], gsem).wait()
        pltpu.make_async_copy(sttgt_hbm.at[sb_tg.at[0]],
                              trow.at[pl.ds(0, C)], gsem).wait()

    def compute(j10, b):
        bC = b * C

        @plsc.parallel_loop(0, C // L, unroll=2)
        def gg_body(gg):
            elanes = lanes + gg * L
            erows = elanes + bC
            spack16 = sb_sp[j10, pl.ds(gg * L, L)]
            rel16 = spack16 & 31
            for h in range(HG):
                hcol = jnp.zeros((L,), jnp.int32) + (c * HG + h)
                a = plsc.load_gather(
                    rows, [erows, jnp.full((L,), 64 + h, jnp.int32)])
                b_ = plsc.load_gather(trow, [erows, hcol])
                sv = a + b_
                sv = jnp.where(sv > 0, sv, 0.2 * sv)
                w1 = jnp.exp(sv)
                w2 = plsc.load_gather(et_v, [rel16, hcol])
                plsc.store_scatter(
                    con0, [elanes, jnp.full((L,), 128 + h, jnp.int32)], w1)
                plsc.store_scatter(
                    con0, [elanes, jnp.full((L,), 132 + h, jnp.int32)], w2)
                for dd in range(DV):
                    d = h * DV + dd
                    vals = plsc.load_gather(
                        rows, [erows, jnp.full((L,), d, jnp.int32)])
                    plsc.store_scatter(
                        con0, [elanes, jnp.full((L,), d, jnp.int32)], w1 * vals)
                    plsc.store_scatter(
                        con0, [elanes, jnp.full((L,), 64 + d, jnp.int32)], w2 * vals)

    fire_idx(0, 0)
    wait_idx()
    fire_gath(0)

    def chunk_body(g, carry):
        j10 = lax.rem(g, 2 * SUP)
        t2 = lax.div(g, 2 * SUP)
        b = lax.rem(g, 2)
        wait_gath()

        @pl.when(j10 == 1)
        def _():
            fire_idx(t2 * 2 + 1, 1)

        @pl.when(j10 == SUP - 1)
        def _():
            wait_idx()

        @pl.when(j10 == SUP + 1)
        def _():
            fire_idx(jnp.minimum(t2 * 2 + 2, NSUPER - 1), 0)

        @pl.when(j10 == 2 * SUP - 1)
        def _():
            wait_idx()

        fire_gath(g + 1)
        compute(j10, b)
        pltpu.sync_copy(con0, acc.at[sb_tg.at[j10]], add=True)
        return carry
    lax.fori_loop(0, NCHUNK, chunk_body, 0)

    wait_gath()
    plsc.subcore_barrier()

    for k in range(NPT // C):
        pltpu.sync_copy(acc.at[pl.ds(nb + k * C, C)],
                        out_hbm.at[c, pl.ds(nb + k * C, C)])
    if rem:
        pltpu.sync_copy(acc.at[pl.ds(nb + (NPT // C) * C, rem)],
                        out_hbm.at[c, pl.ds(nb + (NPT // C) * C, rem)])


_edge_kernel = functools.partial(
    pl.kernel,
    out_type=jax.ShapeDtypeStruct((NC, N, ACCW), jnp.float32),
    mesh=plsc.VectorSubcoreMesh(core_axis_name="c", subcore_axis_name="s"),
    scratch_types=[
        pltpu.VMEM((2 * SUP, C), jnp.int32),
        pltpu.VMEM((2 * SUP, C), jnp.int32),
        pltpu.VMEM((2, C), jnp.int32),
        pltpu.VMEM((2 * C, ROWW), jnp.float32),
        pltpu.VMEM((2 * C, 16), jnp.float32),
        pltpu.VMEM((C, ACCW), jnp.float32),
        pltpu.VMEM((32, H), jnp.float32),
        pltpu.VMEM_SHARED((N, ACCW), jnp.float32),
        pltpu.SemaphoreType.DMA,
        pltpu.SemaphoreType.DMA,
    ],
    compiler_params=pltpu.CompilerParams(use_tc_tiling_on_sc=False,
                                         needs_layout_passes=False),
)(_edge_body)


def _final_body(acc_ref, fpw_ref, fpb_ref, out_ref):
    a = acc_ref[...]
    parts = []
    for hh in range(H):
        cc, j = hh // HG, hh % HG
        den_v = a[cc, :, 128 + j:129 + j] + EPS
        den_r = a[cc, :, 132 + j:133 + j] + EPS
        parts.append(a[cc, :, j * 16:(j + 1) * 16] / den_v)
        parts.append(a[cc, :, 64 + j * 16:64 + (j + 1) * 16] / den_r)
    cat = jnp.concatenate(parts, axis=1)
    out_ref[...] = (jnp.dot(cat, fpw_ref[...], preferred_element_type=jnp.float32)
                    + fpb_ref[...])


def kernel(x, edge_index, rel_pos_idx, W_value, rel_emb, W_relation,
           w_src, w_tgt, w_rel, fp_w, fp_b):
    spack = (edge_index[0] * 32 + rel_pos_idx).reshape(NS, NSUPER, SUP, C)
    tgt_r = edge_index[1].reshape(NS, NSUPER, SUP, C)

    v_all3, st_tgt = pl.pallas_call(
        _prep_body,
        grid=(N // PBLK,),
        in_specs=[
            pl.BlockSpec((PBLK, D), lambda i: (i, 0)),
            pl.BlockSpec((D, D), lambda i: (0, 0)),
            pl.BlockSpec((1, H, DV), lambda i: (0, 0, 0)),
            pl.BlockSpec((1, H, DV), lambda i: (0, 0, 0)),
        ],
        out_specs=[
            pl.BlockSpec((2, PBLK, ROWW), lambda i: (0, i, 0)),
            pl.BlockSpec((PBLK, 16), lambda i: (i, 0)),
        ],
        out_shape=[
            jax.ShapeDtypeStruct((2, N, ROWW), jnp.float32),
            jax.ShapeDtypeStruct((N, 16), jnp.float32),
        ],
    )(x, W_value, w_src, w_tgt)
    v_all = v_all3.reshape(2 * N, ROWW)

    et = pl.pallas_call(
        _tbl_body,
        out_shape=jax.ShapeDtypeStruct((32, H), jnp.float32),
    )(rel_emb, W_relation, w_rel)

    acc2 = _edge_kernel(v_all, st_tgt, et, spack, tgt_r)

    eye = jnp.eye(H, dtype=jnp.float32)
    fpw_bd = (eye[:, None, :, None] * fp_w[:, :, None, :]).reshape(H * 2 * DV, H * DV)
    fpb_flat = fp_b.reshape(1, H * DV)

    BLK = 2000
    return pl.pallas_call(
        _final_body,
        grid=(N // BLK,),
        in_specs=[
            pl.BlockSpec((NC, BLK, ACCW), lambda i: (0, i, 0)),
            pl.BlockSpec((H * 2 * DV, H * DV), lambda i: (0, 0)),
            pl.BlockSpec((1, H * DV), lambda i: (0, 0)),
        ],
        out_specs=pl.BlockSpec((BLK, H * DV), lambda i: (i, 0)),
        out_shape=jax.ShapeDtypeStruct((N, H * DV), jnp.float32),
    )(acc2, fpw_bd, fpb_flat)
